# baseline (speedup 1.0000x reference)
import functools

import jax
import jax.numpy as jnp
from jax import lax
from jax.experimental import pallas as pl
from jax.experimental.pallas import tpu as pltpu
from jax.experimental.pallas import tpu_sc as plsc

N_NODES = 10000
N_EDGES = 320000
D_FEAT = 128

NC = 2
NS = 16
NW = NC * NS
EPW = N_EDGES // NW
ECHUNK = 80
NCHUNK = EPW // ECHUNK
NTRI = (NCHUNK - 2) // 3
ZROWS = ECHUNK
NZBLK = N_NODES // ZROWS
WROWS = 200
NWBLK = N_NODES // WROWS


def _sc_body(dst_hbm, src_hbm, a_hbm, x_hbm, out_hbm,
             src_all, d0, d1, d2, a0, a1, a2, r0, r1, r2,
             acc, isem, gsem0, gsem1, gsem2):
    c = lax.axis_index("c")
    s = lax.axis_index("s")
    wid = c * NS + s
    base = wid * EPW

    dbuf = (d0, d1, d2)
    abuf = (a0, a1, a2)
    rbuf = (r0, r1, r2)
    gsem = (gsem0, gsem1, gsem2)

    hs = pltpu.async_copy(src_hbm.at[pl.ds(base, EPW)], src_all, isem)

    def zrow(i, carry):
        for j in range(D_FEAT // 16):
            r0[i, pl.ds(j * 16, 16)] = jnp.zeros((16,), jnp.float32)
        return carry
    lax.fori_loop(0, ZROWS, zrow, 0)
    for b in range((NZBLK + NS - 1) // NS):
        blk = b * NS + s

        @pl.when(blk < NZBLK)
        def _():
            pltpu.sync_copy(r0, acc.at[pl.ds(blk * ZROWS, ZROWS)])
    hs.wait()
    plsc.subcore_barrier()

    def start_chunk(ci, q3):
        off = base + ci * ECHUNK
        pltpu.async_copy(dst_hbm.at[pl.ds(off, ECHUNK)], dbuf[q3], gsem[q3])
        pltpu.async_copy(a_hbm.at[pl.ds(off, ECHUNK)], abuf[q3], gsem[q3])
        pltpu.async_copy(x_hbm.at[src_all.at[pl.ds(ci * ECHUNK, ECHUNK)]],
                         rbuf[q3], gsem[q3])

    def wait_chunk(q3):
        pltpu.make_async_copy(dst_hbm.at[pl.ds(0, ECHUNK)],
                              dbuf[q3], gsem[q3]).wait()
        pltpu.make_async_copy(a_hbm.at[pl.ds(0, ECHUNK)],
                              abuf[q3], gsem[q3]).wait()
        pltpu.make_async_copy(x_hbm.at[src_all.at[pl.ds(0, ECHUNK)]],
                              rbuf[q3], gsem[q3]).wait()

    def scale(q3):
        r_r, a_r = rbuf[q3], abuf[q3]
        for grp in range(ECHUNK // 16):
            av16 = a_r[pl.ds(grp * 16, 16)]
            for l in range(16):
                a = av16[l]
                e = grp * 16 + l
                for j in range(D_FEAT // 16):
                    sl = pl.ds(j * 16, 16)
                    r_r[e, sl] = r_r[e, sl] * a

    def chunk_step(i, q3):
        wait_chunk(q3)
        scale(q3)
        pltpu.sync_copy(rbuf[q3], acc.at[dbuf[q3]], add=True)

        @pl.when(i + 3 < NCHUNK)
        def _():
            start_chunk(i + 3, q3)

    start_chunk(0, 0)
    start_chunk(1, 1)
    start_chunk(2, 2)

    def tri_body(m, carry):
        chunk_step(3 * m, 0)
        chunk_step(3 * m + 1, 1)
        chunk_step(3 * m + 2, 2)
        return carry
    lax.fori_loop(0, NTRI, tri_body, 0)
    chunk_step(NCHUNK - 2, 0)
    chunk_step(NCHUNK - 1, 1)

    plsc.subcore_barrier()
    for b in range((NWBLK + NS - 1) // NS):
        blk = b * NS + s

        @pl.when(blk < NWBLK)
        def _():
            r = blk * WROWS
            pltpu.sync_copy(acc.at[pl.ds(r, WROWS)],
                            out_hbm.at[c, pl.ds(r, WROWS)])


def _combine_body(p_ref, o_ref):
    o_ref[...] = p_ref[0] + p_ref[1]


def kernel(edge_index, A_values, X):
    mesh = plsc.VectorSubcoreMesh(core_axis_name="c", subcore_axis_name="s")
    sc_call = functools.partial(
        pl.kernel,
        mesh=mesh,
        out_type=jax.ShapeDtypeStruct((NC, N_NODES, D_FEAT), jnp.float32),
        scratch_types=[
            pltpu.VMEM((EPW,), jnp.int32),
            pltpu.VMEM((ECHUNK,), jnp.int32),
            pltpu.VMEM((ECHUNK,), jnp.int32),
            pltpu.VMEM((ECHUNK,), jnp.int32),
            pltpu.VMEM((ECHUNK,), jnp.float32),
            pltpu.VMEM((ECHUNK,), jnp.float32),
            pltpu.VMEM((ECHUNK,), jnp.float32),
            pltpu.VMEM((ECHUNK, D_FEAT), jnp.float32),
            pltpu.VMEM((ECHUNK, D_FEAT), jnp.float32),
            pltpu.VMEM((ECHUNK, D_FEAT), jnp.float32),
            pltpu.VMEM_SHARED((N_NODES, D_FEAT), jnp.float32),
            pltpu.SemaphoreType.DMA,
            pltpu.SemaphoreType.DMA,
            pltpu.SemaphoreType.DMA,
            pltpu.SemaphoreType.DMA,
        ],
    )(_sc_body)
    partials = sc_call(edge_index[0], edge_index[1], A_values, X)

    combine = pl.pallas_call(
        _combine_body,
        out_shape=jax.ShapeDtypeStruct((N_NODES, D_FEAT), jnp.float32),
        grid=(10,),
        in_specs=[pl.BlockSpec((NC, N_NODES // 10, D_FEAT), lambda i: (0, i, 0))],
        out_specs=pl.BlockSpec((N_NODES // 10, D_FEAT), lambda i: (i, 0)),
    )
    return combine(partials)


# speedup vs baseline: 1.3473x; 1.3473x over previous
import functools

import jax
import jax.numpy as jnp
from jax import lax
from jax.experimental import pallas as pl
from jax.experimental.pallas import tpu as pltpu
from jax.experimental.pallas import tpu_sc as plsc

N_NODES = 10000
N_EDGES = 320000
D_FEAT = 128

NC = 2
NS = 16
NW = NC * NS
EPW = N_EDGES // NW
ECHUNK = 80
NCHUNK = EPW // ECHUNK
NPAIR = (NCHUNK - 1) // 2
ZROWS = ECHUNK
NZBLK = N_NODES // ZROWS
WROWS = 200
NWBLK = N_NODES // WROWS


def _sc_body(dst_hbm, src_hbm, a_hbm, x_hbm, out_hbm,
             src_all, a_all, dst0, dst1, rows0, rows1,
             acc, isem, gsem0, gsem1):
    c = lax.axis_index("c")
    s = lax.axis_index("s")
    wid = c * NS + s
    base = wid * EPW

    h1 = pltpu.async_copy(src_hbm.at[pl.ds(base, EPW)], src_all, isem)
    h2 = pltpu.async_copy(a_hbm.at[pl.ds(base, EPW)], a_all, isem)

    def zrow(i, carry):
        for j in range(D_FEAT // 16):
            rows0[i, pl.ds(j * 16, 16)] = jnp.zeros((16,), jnp.float32)
        return carry
    lax.fori_loop(0, ZROWS, zrow, 0)
    for b in range((NZBLK + NS - 1) // NS):
        blk = b * NS + s

        @pl.when(blk < NZBLK)
        def _():
            pltpu.sync_copy(rows0, acc.at[pl.ds(blk * ZROWS, ZROWS)])
    h1.wait()
    h2.wait()
    plsc.subcore_barrier()

    def start_chunk(ci, dst_r, rows_r, sem):
        pltpu.async_copy(dst_hbm.at[pl.ds(base + ci * ECHUNK, ECHUNK)],
                         dst_r, sem)
        pltpu.async_copy(x_hbm.at[src_all.at[pl.ds(ci * ECHUNK, ECHUNK)]],
                         rows_r, sem)

    def wait_chunk(dst_r, rows_r, sem):
        pltpu.make_async_copy(dst_hbm.at[pl.ds(0, ECHUNK)], dst_r, sem).wait()
        pltpu.make_async_copy(x_hbm.at[src_all.at[pl.ds(0, ECHUNK)]],
                              rows_r, sem).wait()

    def scale(ci, rows_r):
        def gbody(g, gcarry):
            av16 = a_all[pl.ds(ci * ECHUNK + g * 16, 16)]
            for l in range(16):
                a = av16[l]
                e = g * 16 + l
                for j in range(D_FEAT // 16):
                    sl = pl.ds(j * 16, 16)
                    rows_r[e, sl] = rows_r[e, sl] * a
            return gcarry
        lax.fori_loop(0, ECHUNK // 16, gbody, 0)

    def scatter_add(dst_r, rows_r):
        pltpu.sync_copy(rows_r, acc.at[dst_r], add=True)

    start_chunk(0, dst0, rows0, gsem0)

    def pair_body(k, carry):
        c0 = 2 * k
        c1 = 2 * k + 1
        start_chunk(c1, dst1, rows1, gsem1)
        wait_chunk(dst0, rows0, gsem0)
        scale(c0, rows0)
        scatter_add(dst0, rows0)
        start_chunk(c0 + 2, dst0, rows0, gsem0)
        wait_chunk(dst1, rows1, gsem1)
        scale(c1, rows1)
        scatter_add(dst1, rows1)
        return carry
    lax.fori_loop(0, NPAIR, pair_body, 0)
    wait_chunk(dst0, rows0, gsem0)
    scale(NCHUNK - 1, rows0)
    scatter_add(dst0, rows0)

    plsc.subcore_barrier()
    for b in range((NWBLK + NS - 1) // NS):
        blk = b * NS + s

        @pl.when(blk < NWBLK)
        def _():
            r = blk * WROWS
            pltpu.sync_copy(acc.at[pl.ds(r, WROWS)],
                            out_hbm.at[c, pl.ds(r, WROWS)])


def _combine_body(p_ref, o_ref):
    o_ref[...] = p_ref[0] + p_ref[1]


def kernel(edge_index, A_values, X):
    mesh = plsc.VectorSubcoreMesh(core_axis_name="c", subcore_axis_name="s")
    sc_call = functools.partial(
        pl.kernel,
        mesh=mesh,
        out_type=jax.ShapeDtypeStruct((NC, N_NODES, D_FEAT), jnp.float32),
        scratch_types=[
            pltpu.VMEM((EPW,), jnp.int32),
            pltpu.VMEM((EPW,), jnp.float32),
            pltpu.VMEM((ECHUNK,), jnp.int32),
            pltpu.VMEM((ECHUNK,), jnp.int32),
            pltpu.VMEM((ECHUNK, D_FEAT), jnp.float32),
            pltpu.VMEM((ECHUNK, D_FEAT), jnp.float32),
            pltpu.VMEM_SHARED((N_NODES, D_FEAT), jnp.float32),
            pltpu.SemaphoreType.DMA,
            pltpu.SemaphoreType.DMA,
            pltpu.SemaphoreType.DMA,
        ],
    )(_sc_body)
    partials = sc_call(edge_index[0], edge_index[1], A_values, X)

    combine = pl.pallas_call(
        _combine_body,
        out_shape=jax.ShapeDtypeStruct((N_NODES, D_FEAT), jnp.float32),
        grid=(10,),
        in_specs=[pl.BlockSpec((NC, N_NODES // 10, D_FEAT), lambda i: (0, i, 0))],
        out_specs=pl.BlockSpec((N_NODES // 10, D_FEAT), lambda i: (i, 0)),
    )
    return combine(partials)
